# Initial kernel scaffold; baseline (speedup 1.0000x reference)
import jax
import jax.numpy as jnp
from jax.experimental import pallas as pl


def kernel(x_p, x_np, y, edge_index_p, edge_index_np, W, b, mlp_w1, mlp_b1, mlp_w2, mlp_b2):
    raise NotImplementedError("write your pallas kernel here")


# speedup vs baseline: 8.6865x; 8.6865x over previous
import functools

import jax
import jax.numpy as jnp
from jax import lax
from jax.experimental import pallas as pl
from jax.experimental.pallas import tpu as pltpu
from jax.experimental.pallas import tpu_sc as plsc

N = 10200
D = 128
E = 326400
BN = 200
NB = 51
CH = 120
NCH = N // CH
CPT = 6
NS = 16
EPT = E // 32

_sc_mesh = plsc.VectorSubcoreMesh(core_axis_name="c", subcore_axis_name="s")


@functools.partial(
    pl.kernel,
    out_type=jax.ShapeDtypeStruct((2, N, D), jnp.float32),
    mesh=_sc_mesh,
    scratch_types=[
        pltpu.VMEM((CH,), jnp.int32),
        pltpu.VMEM((CH,), jnp.int32),
        pltpu.VMEM((CH, D), jnp.float32),
        pltpu.VMEM((CH, D), jnp.float32),
        pltpu.VMEM_SHARED((N, D), jnp.float32),
        pltpu.SemaphoreType.DMA,
    ],
)
def _sc_propagate(h_hbm, src_hbm, dst_hbm, zblk_hbm, out_hbm,
                  sidx_v, didx_v, rows_v, zero_v, acc_sh, sem):
    cid = lax.axis_index("c")
    sid = lax.axis_index("s")
    wid = cid * NS + sid

    pltpu.sync_copy(zblk_hbm, zero_v)

    def zbody(i, carry):
        c = sid * CPT + i

        @pl.when(c < NCH)
        def _():
            pltpu.sync_copy(zero_v, acc_sh.at[pl.ds(c * CH, CH)])
        return carry

    lax.fori_loop(0, CPT, zbody, 0)
    plsc.subcore_barrier()

    base = wid * EPT

    def ebody(j, carry):
        off = base + j * CH
        pltpu.sync_copy(src_hbm.at[pl.ds(off, CH)], sidx_v)
        pltpu.sync_copy(dst_hbm.at[pl.ds(off, CH)], didx_v)
        pltpu.async_copy(h_hbm.at[sidx_v], rows_v, sem).wait()
        pltpu.sync_copy(rows_v, acc_sh.at[didx_v], add=True)
        return carry

    lax.fori_loop(0, NCH, ebody, 0)
    plsc.subcore_barrier()

    def dbody(i, carry):
        c = sid * CPT + i

        @pl.when(c < NCH)
        def _():
            pltpu.sync_copy(acc_sh.at[pl.ds(c * CH, CH)],
                            out_hbm.at[cid, pl.ds(c * CH, CH)])
        return carry

    lax.fori_loop(0, CPT, dbody, 0)


def _dinv_body(d0_ref, d1_ref, out_ref):
    dcol = d0_ref[:, 0:1] + d1_ref[:, 0:1] + 1.0
    out_ref[...] = jnp.broadcast_to(1.0 / jnp.sqrt(dcol), (BN, D))


_tc_dinv = pl.pallas_call(
    _dinv_body,
    grid=(NB,),
    in_specs=[
        pl.BlockSpec((BN, D), lambda g: (g, 0)),
        pl.BlockSpec((BN, D), lambda g: (g, 0)),
    ],
    out_specs=pl.BlockSpec((BN, D), lambda g: (g, 0)),
    out_shape=jax.ShapeDtypeStruct((N, D), jnp.float32),
)


def _init_body(x_ref, w_ref, dinv_ref, out_ref):
    out_ref[...] = dinv_ref[...] * jnp.dot(
        x_ref[...], w_ref[...], preferred_element_type=jnp.float32,
        precision=jax.lax.Precision.HIGHEST)


_tc_init = pl.pallas_call(
    _init_body,
    grid=(NB,),
    in_specs=[
        pl.BlockSpec((BN, D), lambda g: (g, 0)),
        pl.BlockSpec((D, D), lambda g: (0, 0)),
        pl.BlockSpec((BN, D), lambda g: (g, 0)),
    ],
    out_specs=pl.BlockSpec((BN, D), lambda g: (g, 0)),
    out_shape=jax.ShapeDtypeStruct((N, D), jnp.float32),
)


def _block_trace(o):
    r = lax.broadcasted_iota(jnp.int32, (BN, D), 0)
    c = lax.broadcasted_iota(jnp.int32, (BN, D), 1)
    return jnp.sum(jnp.where(r == c, o, 0.0))


def _layer_body(acc0_ref, acc1_ref, h_ref, dinv_ref, w_ref, b_ref,
                hn_ref, tr_ref):
    o = dinv_ref[...] * (acc0_ref[...] + acc1_ref[...] + h_ref[...]) + b_ref[...]
    tr_ref[...] = jnp.full((1, 1, D), _block_trace(o), jnp.float32)
    hn_ref[...] = dinv_ref[...] * jnp.dot(
        o, w_ref[...], preferred_element_type=jnp.float32,
        precision=jax.lax.Precision.HIGHEST)


_tc_layer = pl.pallas_call(
    _layer_body,
    grid=(NB,),
    in_specs=[
        pl.BlockSpec((BN, D), lambda g: (g, 0)),
        pl.BlockSpec((BN, D), lambda g: (g, 0)),
        pl.BlockSpec((BN, D), lambda g: (g, 0)),
        pl.BlockSpec((BN, D), lambda g: (g, 0)),
        pl.BlockSpec((D, D), lambda g: (0, 0)),
        pl.BlockSpec((1, D), lambda g: (0, 0)),
    ],
    out_specs=[
        pl.BlockSpec((BN, D), lambda g: (g, 0)),
        pl.BlockSpec((1, 1, D), lambda g: (g, 0, 0)),
    ],
    out_shape=[
        jax.ShapeDtypeStruct((N, D), jnp.float32),
        jax.ShapeDtypeStruct((NB, 1, D), jnp.float32),
    ],
)


def _last_body(acc0_ref, acc1_ref, h_ref, dinv_ref, b_ref, tr_ref):
    o = dinv_ref[...] * (acc0_ref[...] + acc1_ref[...] + h_ref[...]) + b_ref[...]
    tr_ref[...] = jnp.full((1, 1, D), _block_trace(o), jnp.float32)


_tc_last = pl.pallas_call(
    _last_body,
    grid=(NB,),
    in_specs=[
        pl.BlockSpec((BN, D), lambda g: (g, 0)),
        pl.BlockSpec((BN, D), lambda g: (g, 0)),
        pl.BlockSpec((BN, D), lambda g: (g, 0)),
        pl.BlockSpec((BN, D), lambda g: (g, 0)),
        pl.BlockSpec((1, D), lambda g: (0, 0)),
    ],
    out_specs=pl.BlockSpec((1, 1, D), lambda g: (g, 0, 0)),
    out_shape=jax.ShapeDtypeStruct((NB, 1, D), jnp.float32),
)


def _mlp_body(t_ref, s_ref, w1t_ref, b1_ref, w2t_ref, b2_ref, z_ref):
    t = t_ref[...]
    p = (t - t[:, 50:51]) * s_ref[...]
    mask = lax.broadcasted_iota(jnp.int32, (8, NB), 1) < (NB - 1)
    pm = jnp.where(mask, p, 0.0)
    mu = jnp.sum(pm, axis=1, keepdims=True) / 50.0
    dd = jnp.where(mask, p - mu, 0.0)
    var = jnp.sum(dd * dd, axis=1, keepdims=True) / 49.0
    pn = (p - mu) / jnp.sqrt(var)
    hT = jnp.maximum(jnp.dot(w1t_ref[...], pn,
                             preferred_element_type=jnp.float32,
        precision=jax.lax.Precision.HIGHEST)
                     + b1_ref[...], 0.0)
    zT = jnp.dot(w2t_ref[...], hT, preferred_element_type=jnp.float32,
        precision=jax.lax.Precision.HIGHEST) \
        + b2_ref[...]
    z_ref[...] = 1.0 / (1.0 + jnp.exp(-zT))


_tc_mlp = pl.pallas_call(
    _mlp_body,
    out_shape=jax.ShapeDtypeStruct((1, NB), jnp.float32),
)


def kernel(x_p, x_np, y, edge_index_p, edge_index_np, W, b,
           mlp_w1, mlp_b1, mlp_w2, mlp_b2):
    xc = jnp.concatenate([x_p, x_np], axis=0)
    src = jnp.concatenate([edge_index_p[0], edge_index_np[0] + 10000])
    dst = jnp.concatenate([edge_index_p[1], edge_index_np[1] + 10000])
    src = src.astype(jnp.int32)
    dst = dst.astype(jnp.int32)

    zblk = jnp.zeros((CH, D), jnp.float32)
    onesN = jnp.ones((N, D), jnp.float32)
    b2d = b.reshape(1, D).astype(jnp.float32)

    deg = _sc_propagate(onesN, src, dst, zblk)
    dinv = _tc_dinv(deg[0], deg[1])

    ht = _tc_init(xc, W, dinv)
    traces = []
    for k in range(8):
        acc = _sc_propagate(ht, src, dst, zblk)
        if k < 7:
            ht, tr = _tc_layer(acc[0], acc[1], ht, dinv, W, b2d)
        else:
            tr = _tc_last(acc[0], acc[1], ht, dinv, b2d)
        traces.append(tr[:, 0, 0])

    t = jnp.stack(traces, axis=0)
    s = jnp.concatenate([(y[:, 0] - 0.5) * 2.0,
                         jnp.zeros((1,), jnp.float32)])[None, :]
    zT = _tc_mlp(t, s, mlp_w1.T, mlp_b1.reshape(15, 1),
                 mlp_w2.T, mlp_b2.reshape(1, 1))
    return zT[0, :50].reshape(50, 1)


# speedup vs baseline: 15.7329x; 1.8112x over previous
import functools

import jax
import jax.numpy as jnp
from jax import lax
from jax.experimental import pallas as pl
from jax.experimental.pallas import tpu as pltpu
from jax.experimental.pallas import tpu_sc as plsc

N = 10200
D = 128
E = 326400
BN = 200
NB = 51
CH = 120
NCH = N // CH
CPT = 6
NS = 16
EPT = E // 32

_sc_mesh = plsc.VectorSubcoreMesh(core_axis_name="c", subcore_axis_name="s")


SB = 17
NSB = NCH // SB


@functools.partial(
    pl.kernel,
    out_type=jax.ShapeDtypeStruct((2, N, D), jnp.float32),
    mesh=_sc_mesh,
    scratch_types=[
        pltpu.VMEM((2, SB, 1, CH), jnp.int32),
        pltpu.VMEM((2, SB, 1, CH), jnp.int32),
        pltpu.VMEM((2, CH, D), jnp.float32),
        pltpu.VMEM_SHARED((N, D), jnp.float32),
        pltpu.SemaphoreType.DMA,
        pltpu.SemaphoreType.DMA,
        pltpu.SemaphoreType.DMA,
        pltpu.SemaphoreType.DMA,
    ],
)
def _sc_propagate(h_hbm, src_hbm, dst_hbm, zblk_hbm, out_hbm,
                  sidx_v, didx_v, rows_v, acc_sh,
                  sem0, sem1, semi0, semi1):
    cid = lax.axis_index("c")
    sid = lax.axis_index("s")
    wid = cid * NS + sid
    sems = (sem0, sem1)

    pltpu.async_copy(src_hbm.at[wid, 0], sidx_v.at[0], semi0)
    pltpu.async_copy(dst_hbm.at[wid, 0], didx_v.at[0], semi1)

    pltpu.sync_copy(zblk_hbm, rows_v.at[0])

    def zbody(i, carry):
        c = sid * CPT + i

        @pl.when(c < NCH)
        def _():
            pltpu.sync_copy(rows_v.at[0], acc_sh.at[pl.ds(c * CH, CH)])
        return carry

    lax.fori_loop(0, CPT, zbody, 0)
    plsc.subcore_barrier()

    def sbody(s, carry):
        slot = s % 2
        nslot = 1 - slot
        pltpu.make_async_copy(src_hbm.at[wid, s],
                              sidx_v.at[slot], semi0).wait()
        pltpu.make_async_copy(dst_hbm.at[wid, s],
                              didx_v.at[slot], semi1).wait()

        @pl.when(s + 1 < NSB)
        def _():
            pltpu.async_copy(src_hbm.at[wid, s + 1],
                             sidx_v.at[nslot], semi0)
            pltpu.async_copy(dst_hbm.at[wid, s + 1],
                             didx_v.at[nslot], semi1)

        pltpu.async_copy(h_hbm.at[sidx_v.at[slot, 0, 0]], rows_v.at[0], sem0)
        for q in range(SB):
            if q + 1 < SB:
                pltpu.async_copy(h_hbm.at[sidx_v.at[slot, q + 1, 0]],
                                 rows_v.at[(q + 1) % 2], sems[(q + 1) % 2])
            pltpu.make_async_copy(h_hbm.at[sidx_v.at[slot, q, 0]],
                                  rows_v.at[q % 2], sems[q % 2]).wait()
            pltpu.sync_copy(rows_v.at[q % 2], acc_sh.at[didx_v.at[slot, q, 0]],
                            add=True)
        return carry

    lax.fori_loop(0, NSB, sbody, 0)
    plsc.subcore_barrier()

    def dbody(i, carry):
        c = sid * CPT + i

        @pl.when(c < NCH)
        def _():
            pltpu.sync_copy(acc_sh.at[pl.ds(c * CH, CH)],
                            out_hbm.at[cid, pl.ds(c * CH, CH)])
        return carry

    lax.fori_loop(0, CPT, dbody, 0)


def _dinv_body(d0_ref, d1_ref, out_ref):
    dcol = d0_ref[:, 0:1] + d1_ref[:, 0:1] + 1.0
    out_ref[...] = jnp.broadcast_to(1.0 / jnp.sqrt(dcol), (BN, D))


_tc_dinv = pl.pallas_call(
    _dinv_body,
    grid=(NB,),
    in_specs=[
        pl.BlockSpec((BN, D), lambda g: (g, 0)),
        pl.BlockSpec((BN, D), lambda g: (g, 0)),
    ],
    out_specs=pl.BlockSpec((BN, D), lambda g: (g, 0)),
    out_shape=jax.ShapeDtypeStruct((N, D), jnp.float32),
)


def _init_body(x_ref, w_ref, dinv_ref, out_ref):
    out_ref[...] = dinv_ref[...] * jnp.dot(
        x_ref[...], w_ref[...], preferred_element_type=jnp.float32,
        precision=jax.lax.Precision.HIGHEST)


_tc_init = pl.pallas_call(
    _init_body,
    grid=(NB,),
    in_specs=[
        pl.BlockSpec((BN, D), lambda g: (g, 0)),
        pl.BlockSpec((D, D), lambda g: (0, 0)),
        pl.BlockSpec((BN, D), lambda g: (g, 0)),
    ],
    out_specs=pl.BlockSpec((BN, D), lambda g: (g, 0)),
    out_shape=jax.ShapeDtypeStruct((N, D), jnp.float32),
)


def _block_trace(o):
    r = lax.broadcasted_iota(jnp.int32, (BN, D), 0)
    c = lax.broadcasted_iota(jnp.int32, (BN, D), 1)
    return jnp.sum(jnp.where(r == c, o, 0.0))


def _layer_body(acc0_ref, acc1_ref, h_ref, dinv_ref, w_ref, b_ref,
                hn_ref, tr_ref):
    o = dinv_ref[...] * (acc0_ref[...] + acc1_ref[...] + h_ref[...]) + b_ref[...]
    tr_ref[...] = jnp.full((1, 1, D), _block_trace(o), jnp.float32)
    hn_ref[...] = dinv_ref[...] * jnp.dot(
        o, w_ref[...], preferred_element_type=jnp.float32,
        precision=jax.lax.Precision.HIGHEST)


_tc_layer = pl.pallas_call(
    _layer_body,
    grid=(NB,),
    in_specs=[
        pl.BlockSpec((BN, D), lambda g: (g, 0)),
        pl.BlockSpec((BN, D), lambda g: (g, 0)),
        pl.BlockSpec((BN, D), lambda g: (g, 0)),
        pl.BlockSpec((BN, D), lambda g: (g, 0)),
        pl.BlockSpec((D, D), lambda g: (0, 0)),
        pl.BlockSpec((1, D), lambda g: (0, 0)),
    ],
    out_specs=[
        pl.BlockSpec((BN, D), lambda g: (g, 0)),
        pl.BlockSpec((1, 1, D), lambda g: (g, 0, 0)),
    ],
    out_shape=[
        jax.ShapeDtypeStruct((N, D), jnp.float32),
        jax.ShapeDtypeStruct((NB, 1, D), jnp.float32),
    ],
)


def _last_body(acc0_ref, acc1_ref, h_ref, dinv_ref, b_ref, tr_ref):
    o = dinv_ref[...] * (acc0_ref[...] + acc1_ref[...] + h_ref[...]) + b_ref[...]
    tr_ref[...] = jnp.full((1, 1, D), _block_trace(o), jnp.float32)


_tc_last = pl.pallas_call(
    _last_body,
    grid=(NB,),
    in_specs=[
        pl.BlockSpec((BN, D), lambda g: (g, 0)),
        pl.BlockSpec((BN, D), lambda g: (g, 0)),
        pl.BlockSpec((BN, D), lambda g: (g, 0)),
        pl.BlockSpec((BN, D), lambda g: (g, 0)),
        pl.BlockSpec((1, D), lambda g: (0, 0)),
    ],
    out_specs=pl.BlockSpec((1, 1, D), lambda g: (g, 0, 0)),
    out_shape=jax.ShapeDtypeStruct((NB, 1, D), jnp.float32),
)


def _mlp_body(t_ref, s_ref, w1t_ref, b1_ref, w2t_ref, b2_ref, z_ref):
    t = t_ref[...]
    p = (t - t[:, 50:51]) * s_ref[...]
    mask = lax.broadcasted_iota(jnp.int32, (8, NB), 1) < (NB - 1)
    pm = jnp.where(mask, p, 0.0)
    mu = jnp.sum(pm, axis=1, keepdims=True) / 50.0
    dd = jnp.where(mask, p - mu, 0.0)
    var = jnp.sum(dd * dd, axis=1, keepdims=True) / 49.0
    pn = (p - mu) / jnp.sqrt(var)
    hT = jnp.maximum(jnp.dot(w1t_ref[...], pn,
                             preferred_element_type=jnp.float32,
        precision=jax.lax.Precision.HIGHEST)
                     + b1_ref[...], 0.0)
    zT = jnp.dot(w2t_ref[...], hT, preferred_element_type=jnp.float32,
        precision=jax.lax.Precision.HIGHEST) \
        + b2_ref[...]
    z_ref[...] = 1.0 / (1.0 + jnp.exp(-zT))


_tc_mlp = pl.pallas_call(
    _mlp_body,
    out_shape=jax.ShapeDtypeStruct((1, NB), jnp.float32),
)


def kernel(x_p, x_np, y, edge_index_p, edge_index_np, W, b,
           mlp_w1, mlp_b1, mlp_w2, mlp_b2):
    xc = jnp.concatenate([x_p, x_np], axis=0)
    src = jnp.concatenate([edge_index_p[0], edge_index_np[0] + 10000])
    dst = jnp.concatenate([edge_index_p[1], edge_index_np[1] + 10000])
    src = src.astype(jnp.int32).reshape(32, NSB, SB, 1, CH)
    dst = dst.astype(jnp.int32).reshape(32, NSB, SB, 1, CH)

    zblk = jnp.zeros((CH, D), jnp.float32)
    onesN = jnp.ones((N, D), jnp.float32)
    b2d = b.reshape(1, D).astype(jnp.float32)

    deg = _sc_propagate(onesN, src, dst, zblk)
    dinv = _tc_dinv(deg[0], deg[1])

    ht = _tc_init(xc, W, dinv)
    traces = []
    for k in range(8):
        acc = _sc_propagate(ht, src, dst, zblk)
        if k < 7:
            ht, tr = _tc_layer(acc[0], acc[1], ht, dinv, W, b2d)
        else:
            tr = _tc_last(acc[0], acc[1], ht, dinv, b2d)
        traces.append(tr[:, 0, 0])

    t = jnp.stack(traces, axis=0)
    s = jnp.concatenate([(y[:, 0] - 0.5) * 2.0,
                         jnp.zeros((1,), jnp.float32)])[None, :]
    zT = _tc_mlp(t, s, mlp_w1.T, mlp_b1.reshape(15, 1),
                 mlp_w2.T, mlp_b2.reshape(1, 1))
    return zT[0, :50].reshape(50, 1)


# speedup vs baseline: 19.3937x; 1.2327x over previous
import functools

import jax
import jax.numpy as jnp
from jax import lax
from jax.experimental import pallas as pl
from jax.experimental.pallas import tpu as pltpu
from jax.experimental.pallas import tpu_sc as plsc

N = 10200
D = 128
E = 326400
BN = 200
BR = 3400
GB = BR // BN
NTB = N // BR
NB = 51
CH = 120
NCH = N // CH
CPT = 6
NS = 16
SB = 17
NSB = NCH // SB

_sc_mesh = plsc.VectorSubcoreMesh(core_axis_name="c", subcore_axis_name="s")


def _sc_zero_acc(sid, zsrc_v, acc_sh):
    def zbody(i, carry):
        c = sid * CPT + i

        @pl.when(c < NCH)
        def _():
            pltpu.sync_copy(zsrc_v, acc_sh.at[pl.ds(c * CH, CH)])
        return carry

    lax.fori_loop(0, CPT, zbody, 0)


def _sc_drain_acc(cid, sid, acc_sh, out_hbm):
    def dbody(i, carry):
        c = sid * CPT + i

        @pl.when(c < NCH)
        def _():
            pltpu.sync_copy(acc_sh.at[pl.ds(c * CH, CH)],
                            out_hbm.at[cid, pl.ds(c * CH, CH)])
        return carry

    lax.fori_loop(0, CPT, dbody, 0)


@functools.partial(
    pl.kernel,
    out_type=jax.ShapeDtypeStruct((2, N, D), jnp.float32),
    mesh=_sc_mesh,
    scratch_types=[
        pltpu.VMEM((2, SB, 1, CH), jnp.int32),
        pltpu.VMEM((2, SB, 1, CH), jnp.int32),
        pltpu.VMEM((2, CH, D), jnp.float32),
        pltpu.VMEM_SHARED((N, D), jnp.float32),
        pltpu.SemaphoreType.DMA,
        pltpu.SemaphoreType.DMA,
        pltpu.SemaphoreType.DMA,
        pltpu.SemaphoreType.DMA,
        pltpu.SemaphoreType.DMA,
        pltpu.SemaphoreType.DMA,
    ],
)
def _sc_propagate(h_hbm, src_hbm, dst_hbm, zblk_hbm, out_hbm,
                  sidx_v, didx_v, rows_v, acc_sh,
                  semg0, semg1, sems0, sems1, semi0, semi1):
    cid = lax.axis_index("c")
    sid = lax.axis_index("s")
    wid = cid * NS + sid
    semg = (semg0, semg1)
    sems = (sems0, sems1)

    pltpu.async_copy(src_hbm.at[wid, 0], sidx_v.at[0], semi0)
    pltpu.async_copy(dst_hbm.at[wid, 0], didx_v.at[0], semi1)

    pltpu.sync_copy(zblk_hbm, rows_v.at[0])
    _sc_zero_acc(sid, rows_v.at[0], acc_sh)
    plsc.subcore_barrier()

    def sbody(s, carry):
        slot = s % 2
        nslot = 1 - slot
        pltpu.make_async_copy(src_hbm.at[wid, s],
                              sidx_v.at[slot], semi0).wait()
        pltpu.make_async_copy(dst_hbm.at[wid, s],
                              didx_v.at[slot], semi1).wait()

        @pl.when(s + 1 < NSB)
        def _():
            pltpu.async_copy(src_hbm.at[wid, s + 1],
                             sidx_v.at[nslot], semi0)
            pltpu.async_copy(dst_hbm.at[wid, s + 1],
                             didx_v.at[nslot], semi1)

        pltpu.async_copy(h_hbm.at[sidx_v.at[slot, 0, 0]], rows_v.at[0], semg0)
        for q in range(SB):
            b = q % 2
            nb = (q + 1) % 2
            if q + 1 < SB:
                if q >= 1:
                    pltpu.make_async_copy(
                        rows_v.at[nb], acc_sh.at[didx_v.at[slot, q - 1, 0]],
                        sems[nb]).wait()
                pltpu.async_copy(h_hbm.at[sidx_v.at[slot, q + 1, 0]],
                                 rows_v.at[nb], semg[nb])
            pltpu.make_async_copy(h_hbm.at[sidx_v.at[slot, q, 0]],
                                  rows_v.at[b], semg[b]).wait()
            pltpu.async_copy(rows_v.at[b], acc_sh.at[didx_v.at[slot, q, 0]],
                             sems[b], add=True)
        pltpu.make_async_copy(rows_v.at[(SB - 2) % 2],
                              acc_sh.at[didx_v.at[slot, SB - 2, 0]],
                              sems[(SB - 2) % 2]).wait()
        pltpu.make_async_copy(rows_v.at[(SB - 1) % 2],
                              acc_sh.at[didx_v.at[slot, SB - 1, 0]],
                              sems[(SB - 1) % 2]).wait()
        return carry

    lax.fori_loop(0, NSB, sbody, 0)
    plsc.subcore_barrier()
    _sc_drain_acc(cid, sid, acc_sh, out_hbm)


@functools.partial(
    pl.kernel,
    out_type=jax.ShapeDtypeStruct((2, N, D), jnp.float32),
    mesh=_sc_mesh,
    scratch_types=[
        pltpu.VMEM((2, SB, 1, CH), jnp.int32),
        pltpu.VMEM((CH, D), jnp.float32),
        pltpu.VMEM_SHARED((N, D), jnp.float32),
        pltpu.SemaphoreType.DMA,
        pltpu.SemaphoreType.DMA,
    ],
)
def _sc_hist(dst_hbm, zblk_hbm, ones_hbm, out_hbm,
             didx_v, ones_v, acc_sh, sems, semi):
    cid = lax.axis_index("c")
    sid = lax.axis_index("s")
    wid = cid * NS + sid

    pltpu.async_copy(dst_hbm.at[wid, 0], didx_v.at[0], semi)
    pltpu.sync_copy(zblk_hbm, ones_v)
    _sc_zero_acc(sid, ones_v, acc_sh)
    pltpu.sync_copy(ones_hbm, ones_v)
    plsc.subcore_barrier()

    def sbody(s, carry):
        slot = s % 2
        pltpu.make_async_copy(dst_hbm.at[wid, s],
                              didx_v.at[slot], semi).wait()

        @pl.when(s + 1 < NSB)
        def _():
            pltpu.async_copy(dst_hbm.at[wid, s + 1],
                             didx_v.at[1 - slot], semi)

        for q in range(SB):
            pltpu.async_copy(ones_v, acc_sh.at[didx_v.at[slot, q, 0]],
                             sems, add=True)
        for q in range(SB):
            pltpu.make_async_copy(ones_v, acc_sh.at[didx_v.at[slot, q, 0]],
                                  sems).wait()
        return carry

    lax.fori_loop(0, NSB, sbody, 0)
    plsc.subcore_barrier()
    _sc_drain_acc(cid, sid, acc_sh, out_hbm)


def _traces_of(o):
    r = lax.broadcasted_iota(jnp.int32, (BR, D), 0)
    c = lax.broadcasted_iota(jnp.int32, (BR, D), 1)
    masked = jnp.where(r % BN == c, o, 0.0)
    t = jnp.sum(masked.reshape(GB, BN, D), axis=(1, 2))
    return jnp.broadcast_to(t[:, None, None], (GB, 1, D))


def _init_body(x_ref, w_ref, d0_ref, d1_ref, ht_ref, dinv_ref):
    dcol = d0_ref[:, 0:1] + d1_ref[:, 0:1] + 1.0
    dinv = jnp.broadcast_to(1.0 / jnp.sqrt(dcol), (BR, D))
    dinv_ref[...] = dinv
    ht_ref[...] = dinv * jnp.dot(
        x_ref[...], w_ref[...], preferred_element_type=jnp.float32,
        precision=jax.lax.Precision.HIGHEST)


_tc_init = pl.pallas_call(
    _init_body,
    grid=(NTB,),
    in_specs=[
        pl.BlockSpec((BR, D), lambda g: (g, 0)),
        pl.BlockSpec((D, D), lambda g: (0, 0)),
        pl.BlockSpec((BR, D), lambda g: (g, 0)),
        pl.BlockSpec((BR, D), lambda g: (g, 0)),
    ],
    out_specs=[
        pl.BlockSpec((BR, D), lambda g: (g, 0)),
        pl.BlockSpec((BR, D), lambda g: (g, 0)),
    ],
    out_shape=[
        jax.ShapeDtypeStruct((N, D), jnp.float32),
        jax.ShapeDtypeStruct((N, D), jnp.float32),
    ],
)


def _layer_body(acc0_ref, acc1_ref, h_ref, dinv_ref, w_ref, b_ref,
                hn_ref, tr_ref):
    o = dinv_ref[...] * (acc0_ref[...] + acc1_ref[...] + h_ref[...]) + b_ref[...]
    tr_ref[...] = _traces_of(o)
    hn_ref[...] = dinv_ref[...] * jnp.dot(
        o, w_ref[...], preferred_element_type=jnp.float32,
        precision=jax.lax.Precision.HIGHEST)


_tc_layer = pl.pallas_call(
    _layer_body,
    grid=(NTB,),
    in_specs=[
        pl.BlockSpec((BR, D), lambda g: (g, 0)),
        pl.BlockSpec((BR, D), lambda g: (g, 0)),
        pl.BlockSpec((BR, D), lambda g: (g, 0)),
        pl.BlockSpec((BR, D), lambda g: (g, 0)),
        pl.BlockSpec((D, D), lambda g: (0, 0)),
        pl.BlockSpec((1, D), lambda g: (0, 0)),
    ],
    out_specs=[
        pl.BlockSpec((BR, D), lambda g: (g, 0)),
        pl.BlockSpec((GB, 1, D), lambda g: (g, 0, 0)),
    ],
    out_shape=[
        jax.ShapeDtypeStruct((N, D), jnp.float32),
        jax.ShapeDtypeStruct((NB, 1, D), jnp.float32),
    ],
)


def _last_body(acc0_ref, acc1_ref, h_ref, dinv_ref, b_ref, tr_ref):
    o = dinv_ref[...] * (acc0_ref[...] + acc1_ref[...] + h_ref[...]) + b_ref[...]
    tr_ref[...] = _traces_of(o)


_tc_last = pl.pallas_call(
    _last_body,
    grid=(NTB,),
    in_specs=[
        pl.BlockSpec((BR, D), lambda g: (g, 0)),
        pl.BlockSpec((BR, D), lambda g: (g, 0)),
        pl.BlockSpec((BR, D), lambda g: (g, 0)),
        pl.BlockSpec((BR, D), lambda g: (g, 0)),
        pl.BlockSpec((1, D), lambda g: (0, 0)),
    ],
    out_specs=pl.BlockSpec((GB, 1, D), lambda g: (g, 0, 0)),
    out_shape=jax.ShapeDtypeStruct((NB, 1, D), jnp.float32),
)


def _mlp_body(t_ref, s_ref, w1t_ref, b1_ref, w2t_ref, b2_ref, z_ref):
    t = t_ref[...]
    p = (t - t[:, 50:51]) * s_ref[...]
    mask = lax.broadcasted_iota(jnp.int32, (8, NB), 1) < (NB - 1)
    pm = jnp.where(mask, p, 0.0)
    mu = jnp.sum(pm, axis=1, keepdims=True) / 50.0
    dd = jnp.where(mask, p - mu, 0.0)
    var = jnp.sum(dd * dd, axis=1, keepdims=True) / 49.0
    pn = (p - mu) / jnp.sqrt(var)
    hT = jnp.maximum(jnp.dot(w1t_ref[...], pn,
                             preferred_element_type=jnp.float32,
                             precision=jax.lax.Precision.HIGHEST)
                     + b1_ref[...], 0.0)
    zT = jnp.dot(w2t_ref[...], hT, preferred_element_type=jnp.float32,
                 precision=jax.lax.Precision.HIGHEST) + b2_ref[...]
    z_ref[...] = 1.0 / (1.0 + jnp.exp(-zT))


_tc_mlp = pl.pallas_call(
    _mlp_body,
    out_shape=jax.ShapeDtypeStruct((1, NB), jnp.float32),
)


def kernel(x_p, x_np, y, edge_index_p, edge_index_np, W, b,
           mlp_w1, mlp_b1, mlp_w2, mlp_b2):
    xc = jnp.concatenate([x_p, x_np], axis=0)
    src = jnp.concatenate([edge_index_p[0], edge_index_np[0] + 10000])
    dst = jnp.concatenate([edge_index_p[1], edge_index_np[1] + 10000])
    src = src.astype(jnp.int32).reshape(32, NSB, SB, 1, CH)
    dst = dst.astype(jnp.int32).reshape(32, NSB, SB, 1, CH)

    zblk = jnp.zeros((CH, D), jnp.float32)
    onesblk = jnp.ones((CH, D), jnp.float32)
    b2d = b.reshape(1, D).astype(jnp.float32)

    deg = _sc_hist(dst, zblk, onesblk)
    ht, dinv = _tc_init(xc, W, deg[0], deg[1])

    traces = []
    for k in range(8):
        acc = _sc_propagate(ht, src, dst, zblk)
        if k < 7:
            ht, tr = _tc_layer(acc[0], acc[1], ht, dinv, W, b2d)
        else:
            tr = _tc_last(acc[0], acc[1], ht, dinv, b2d)
        traces.append(tr[:, 0, 0])

    t = jnp.stack(traces, axis=0)
    s = jnp.concatenate([(y[:, 0] - 0.5) * 2.0,
                         jnp.zeros((1,), jnp.float32)])[None, :]
    zT = _tc_mlp(t, s, mlp_w1.T, mlp_b1.reshape(15, 1),
                 mlp_w2.T, mlp_b2.reshape(1, 1))
    return zT[0, :50].reshape(50, 1)


# speedup vs baseline: 19.8824x; 1.0252x over previous
import functools

import jax
import jax.numpy as jnp
from jax import lax
from jax.experimental import pallas as pl
from jax.experimental.pallas import tpu as pltpu
from jax.experimental.pallas import tpu_sc as plsc

N = 10200
D = 128
E = 326400
BN = 200
BR = 3400
GB = BR // BN
NTB = N // BR
NB = 51
CH = 120
NCH = N // CH
CPT = 6
NS = 16
SB = 17
NSB = NCH // SB

_sc_mesh = plsc.VectorSubcoreMesh(core_axis_name="c", subcore_axis_name="s")


def _sc_zero_acc(sid, zsrc_v, acc_sh):
    def zbody(i, carry):
        c = sid * CPT + i

        @pl.when(c < NCH)
        def _():
            pltpu.sync_copy(zsrc_v, acc_sh.at[pl.ds(c * CH, CH)])
        return carry

    lax.fori_loop(0, CPT, zbody, 0)


def _sc_drain_acc(cid, sid, acc_sh, out_hbm):
    def dbody(i, carry):
        c = sid * CPT + i

        @pl.when(c < NCH)
        def _():
            pltpu.sync_copy(acc_sh.at[pl.ds(c * CH, CH)],
                            out_hbm.at[cid, pl.ds(c * CH, CH)])
        return carry

    lax.fori_loop(0, CPT, dbody, 0)


@functools.partial(
    pl.kernel,
    out_type=jax.ShapeDtypeStruct((2, N, D), jnp.float32),
    mesh=_sc_mesh,
    scratch_types=[
        pltpu.VMEM((2, SB, 1, CH), jnp.int32),
        pltpu.VMEM((2, SB, 1, CH), jnp.int32),
        pltpu.VMEM((2, CH, D), jnp.float32),
        pltpu.VMEM_SHARED((N, D), jnp.float32),
        pltpu.SemaphoreType.DMA,
        pltpu.SemaphoreType.DMA,
        pltpu.SemaphoreType.DMA,
        pltpu.SemaphoreType.DMA,
        pltpu.SemaphoreType.DMA,
        pltpu.SemaphoreType.DMA,
    ],
)
def _sc_propagate(h_hbm, src_hbm, dst_hbm, zblk_hbm, out_hbm,
                  sidx_v, didx_v, rows_v, acc_sh,
                  semg0, semg1, sems0, sems1, semi0, semi1):
    cid = lax.axis_index("c")
    sid = lax.axis_index("s")
    wid = cid * NS + sid
    semg = (semg0, semg1)
    sems = (sems0, sems1)

    pltpu.async_copy(src_hbm.at[wid, 0], sidx_v.at[0], semi0)
    pltpu.async_copy(dst_hbm.at[wid, 0], didx_v.at[0], semi1)

    pltpu.sync_copy(zblk_hbm, rows_v.at[0])
    _sc_zero_acc(sid, rows_v.at[0], acc_sh)
    plsc.subcore_barrier()

    def sbody(s, carry):
        slot = s % 2
        nslot = 1 - slot
        pltpu.make_async_copy(src_hbm.at[wid, s],
                              sidx_v.at[slot], semi0).wait()
        pltpu.make_async_copy(dst_hbm.at[wid, s],
                              didx_v.at[slot], semi1).wait()

        @pl.when(s + 1 < NSB)
        def _():
            pltpu.async_copy(src_hbm.at[wid, s + 1],
                             sidx_v.at[nslot], semi0)
            pltpu.async_copy(dst_hbm.at[wid, s + 1],
                             didx_v.at[nslot], semi1)

        pltpu.async_copy(h_hbm.at[sidx_v.at[slot, 0, 0]], rows_v.at[0], semg0)
        for q in range(SB):
            b = q % 2
            nb = (q + 1) % 2
            if q + 1 < SB:
                if q >= 1:
                    pltpu.make_async_copy(
                        rows_v.at[nb], acc_sh.at[didx_v.at[slot, q - 1, 0]],
                        sems[nb]).wait()
                pltpu.async_copy(h_hbm.at[sidx_v.at[slot, q + 1, 0]],
                                 rows_v.at[nb], semg[nb])
            pltpu.make_async_copy(h_hbm.at[sidx_v.at[slot, q, 0]],
                                  rows_v.at[b], semg[b]).wait()
            pltpu.async_copy(rows_v.at[b], acc_sh.at[didx_v.at[slot, q, 0]],
                             sems[b], add=True)
        pltpu.make_async_copy(rows_v.at[(SB - 2) % 2],
                              acc_sh.at[didx_v.at[slot, SB - 2, 0]],
                              sems[(SB - 2) % 2]).wait()
        pltpu.make_async_copy(rows_v.at[(SB - 1) % 2],
                              acc_sh.at[didx_v.at[slot, SB - 1, 0]],
                              sems[(SB - 1) % 2]).wait()
        return carry

    lax.fori_loop(0, NSB, sbody, 0)
    plsc.subcore_barrier()
    _sc_drain_acc(cid, sid, acc_sh, out_hbm)


@functools.partial(
    pl.kernel,
    out_type=jax.ShapeDtypeStruct((2, N, D), jnp.float32),
    mesh=_sc_mesh,
    scratch_types=[
        pltpu.VMEM((2, SB, 1, CH), jnp.int32),
        pltpu.VMEM((CH, D), jnp.float32),
        pltpu.VMEM_SHARED((N, D), jnp.float32),
        pltpu.SemaphoreType.DMA,
        pltpu.SemaphoreType.DMA,
    ],
)
def _sc_hist(dst_hbm, zblk_hbm, ones_hbm, out_hbm,
             didx_v, ones_v, acc_sh, sems, semi):
    cid = lax.axis_index("c")
    sid = lax.axis_index("s")
    wid = cid * NS + sid

    pltpu.async_copy(dst_hbm.at[wid, 0], didx_v.at[0], semi)
    pltpu.sync_copy(zblk_hbm, ones_v)
    _sc_zero_acc(sid, ones_v, acc_sh)
    pltpu.sync_copy(ones_hbm, ones_v)
    plsc.subcore_barrier()

    def sbody(s, carry):
        slot = s % 2
        pltpu.make_async_copy(dst_hbm.at[wid, s],
                              didx_v.at[slot], semi).wait()

        @pl.when(s + 1 < NSB)
        def _():
            pltpu.async_copy(dst_hbm.at[wid, s + 1],
                             didx_v.at[1 - slot], semi)

        for q in range(SB):
            pltpu.async_copy(ones_v, acc_sh.at[didx_v.at[slot, q, 0]],
                             sems, add=True)
        for q in range(SB):
            pltpu.make_async_copy(ones_v, acc_sh.at[didx_v.at[slot, q, 0]],
                                  sems).wait()
        return carry

    lax.fori_loop(0, NSB, sbody, 0)
    plsc.subcore_barrier()
    _sc_drain_acc(cid, sid, acc_sh, out_hbm)


def _traces_of(o):
    r = lax.broadcasted_iota(jnp.int32, (BR, D), 0)
    c = lax.broadcasted_iota(jnp.int32, (BR, D), 1)
    masked = jnp.where(r % BN == c, o, 0.0)
    t = jnp.sum(masked.reshape(GB, BN, D), axis=(1, 2))
    return jnp.broadcast_to(t[:, None, None], (GB, 1, D))


def _init_body(x_ref, w_ref, d0_ref, d1_ref, ht_ref, dinv_ref):
    dcol = d0_ref[:, 0:1] + d1_ref[:, 0:1] + 1.0
    dinv = jnp.broadcast_to(1.0 / jnp.sqrt(dcol), (BR, D))
    dinv_ref[...] = dinv
    ht_ref[...] = dinv * jnp.dot(
        x_ref[...], w_ref[...], preferred_element_type=jnp.float32)


_tc_init = pl.pallas_call(
    _init_body,
    grid=(NTB,),
    in_specs=[
        pl.BlockSpec((BR, D), lambda g: (g, 0)),
        pl.BlockSpec((D, D), lambda g: (0, 0)),
        pl.BlockSpec((BR, D), lambda g: (g, 0)),
        pl.BlockSpec((BR, D), lambda g: (g, 0)),
    ],
    out_specs=[
        pl.BlockSpec((BR, D), lambda g: (g, 0)),
        pl.BlockSpec((BR, D), lambda g: (g, 0)),
    ],
    out_shape=[
        jax.ShapeDtypeStruct((N, D), jnp.float32),
        jax.ShapeDtypeStruct((N, D), jnp.float32),
    ],
)


def _layer_body(acc0_ref, acc1_ref, h_ref, dinv_ref, w_ref, b_ref,
                hn_ref, tr_ref):
    o = dinv_ref[...] * (acc0_ref[...] + acc1_ref[...] + h_ref[...]) + b_ref[...]
    tr_ref[...] = _traces_of(o)
    hn_ref[...] = dinv_ref[...] * jnp.dot(
        o, w_ref[...], preferred_element_type=jnp.float32)


_tc_layer = pl.pallas_call(
    _layer_body,
    grid=(NTB,),
    in_specs=[
        pl.BlockSpec((BR, D), lambda g: (g, 0)),
        pl.BlockSpec((BR, D), lambda g: (g, 0)),
        pl.BlockSpec((BR, D), lambda g: (g, 0)),
        pl.BlockSpec((BR, D), lambda g: (g, 0)),
        pl.BlockSpec((D, D), lambda g: (0, 0)),
        pl.BlockSpec((1, D), lambda g: (0, 0)),
    ],
    out_specs=[
        pl.BlockSpec((BR, D), lambda g: (g, 0)),
        pl.BlockSpec((GB, 1, D), lambda g: (g, 0, 0)),
    ],
    out_shape=[
        jax.ShapeDtypeStruct((N, D), jnp.float32),
        jax.ShapeDtypeStruct((NB, 1, D), jnp.float32),
    ],
)


def _last_body(acc0_ref, acc1_ref, h_ref, dinv_ref, b_ref, tr_ref):
    o = dinv_ref[...] * (acc0_ref[...] + acc1_ref[...] + h_ref[...]) + b_ref[...]
    tr_ref[...] = _traces_of(o)


_tc_last = pl.pallas_call(
    _last_body,
    grid=(NTB,),
    in_specs=[
        pl.BlockSpec((BR, D), lambda g: (g, 0)),
        pl.BlockSpec((BR, D), lambda g: (g, 0)),
        pl.BlockSpec((BR, D), lambda g: (g, 0)),
        pl.BlockSpec((BR, D), lambda g: (g, 0)),
        pl.BlockSpec((1, D), lambda g: (0, 0)),
    ],
    out_specs=pl.BlockSpec((GB, 1, D), lambda g: (g, 0, 0)),
    out_shape=jax.ShapeDtypeStruct((NB, 1, D), jnp.float32),
)


def _mlp_body(t_ref, s_ref, w1t_ref, b1_ref, w2t_ref, b2_ref, z_ref):
    t = t_ref[...]
    p = (t - t[:, 50:51]) * s_ref[...]
    mask = lax.broadcasted_iota(jnp.int32, (8, NB), 1) < (NB - 1)
    pm = jnp.where(mask, p, 0.0)
    mu = jnp.sum(pm, axis=1, keepdims=True) / 50.0
    dd = jnp.where(mask, p - mu, 0.0)
    var = jnp.sum(dd * dd, axis=1, keepdims=True) / 49.0
    pn = (p - mu) / jnp.sqrt(var)
    hT = jnp.maximum(jnp.dot(w1t_ref[...], pn,
                             preferred_element_type=jnp.float32,
                             precision=jax.lax.Precision.HIGHEST)
                     + b1_ref[...], 0.0)
    zT = jnp.dot(w2t_ref[...], hT, preferred_element_type=jnp.float32,
                 precision=jax.lax.Precision.HIGHEST) + b2_ref[...]
    z_ref[...] = 1.0 / (1.0 + jnp.exp(-zT))


_tc_mlp = pl.pallas_call(
    _mlp_body,
    out_shape=jax.ShapeDtypeStruct((1, NB), jnp.float32),
)


def kernel(x_p, x_np, y, edge_index_p, edge_index_np, W, b,
           mlp_w1, mlp_b1, mlp_w2, mlp_b2):
    xc = jnp.concatenate([x_p, x_np], axis=0)
    src = jnp.concatenate([edge_index_p[0], edge_index_np[0] + 10000])
    dst = jnp.concatenate([edge_index_p[1], edge_index_np[1] + 10000])
    src = src.astype(jnp.int32).reshape(32, NSB, SB, 1, CH)
    dst = dst.astype(jnp.int32).reshape(32, NSB, SB, 1, CH)

    zblk = jnp.zeros((CH, D), jnp.float32)
    onesblk = jnp.ones((CH, D), jnp.float32)
    b2d = b.reshape(1, D).astype(jnp.float32)

    deg = _sc_hist(dst, zblk, onesblk)
    ht, dinv = _tc_init(xc, W, deg[0], deg[1])

    traces = []
    for k in range(8):
        acc = _sc_propagate(ht, src, dst, zblk)
        if k < 7:
            ht, tr = _tc_layer(acc[0], acc[1], ht, dinv, W, b2d)
        else:
            tr = _tc_last(acc[0], acc[1], ht, dinv, b2d)
        traces.append(tr[:, 0, 0])

    t = jnp.stack(traces, axis=0)
    s = jnp.concatenate([(y[:, 0] - 0.5) * 2.0,
                         jnp.zeros((1,), jnp.float32)])[None, :]
    zT = _tc_mlp(t, s, mlp_w1.T, mlp_b1.reshape(15, 1),
                 mlp_w2.T, mlp_b2.reshape(1, 1))
    return zT[0, :50].reshape(50, 1)
